# Initial kernel scaffold; baseline (speedup 1.0000x reference)
import jax
import jax.numpy as jnp
from jax.experimental import pallas as pl


def kernel(pred0, pred1, masks0, masks1):
    raise NotImplementedError("write your pallas kernel here")


# speedup vs baseline: 5.3777x; 5.3777x over previous
import functools

import jax
import jax.numpy as jnp
from jax import lax
from jax.experimental import pallas as pl
from jax.experimental.pallas import tpu as pltpu
from jax.experimental.pallas import tpu_sc as plsc

B, L, N, C, M = 2, 10, 8192, 100, 32
NCORES, NSUB = 2, 16
NW = NCORES * NSUB
P = N // NW
NCOMBO = 2 * B
ROWS = NCOMBO * L * M
CTAIL = 84


def _sc_kernel_body(pred0, pred1, masks0, masks1, zeros_acc, zeros_cnt,
                    sums_out, cnt_out,
                    mbuf, idx_ref, pbuf, acc, cnt):
    cid = lax.axis_index("c")
    sid = lax.axis_index("s")
    wid = sid * NCORES + cid
    p0 = wid * P

    pltpu.sync_copy(zeros_cnt, cnt)

    iota = jax.lax.broadcasted_iota(jnp.int32, (16,), 0)
    tail_keep = iota >= (2 * 16 - (C - CTAIL))
    ones16 = jnp.ones((16,), jnp.float32)

    masks = (masks0, masks1)
    for combo in range(NCOMBO):
        b, f = combo // 2, combo % 2
        pltpu.sync_copy(masks[f].at[b, :, pl.ds(p0, P)], mbuf)

        def _group(g, _):
            col = g * 16
            best = mbuf[0, pl.ds(col, 16)]
            bidx = jnp.zeros((16,), jnp.int32)

            def _scan_m(m, carry):
                best, bidx = carry
                v = mbuf[m, pl.ds(col, 16)]
                gt = v > best
                bidx = jnp.where(gt, jnp.full((16,), 1, jnp.int32) * m, bidx)
                best = jnp.maximum(v, best)
                return best, bidx

            _, bidx = lax.fori_loop(1, M, _scan_m, (best, bidx))
            idx_ref[combo * 2 + g // 8, pl.ds((g % 8) * 16, 16)] = bidx
            return 0

        lax.fori_loop(0, P // 16, _group, 0)

    preds = (pred0, pred1)
    for combo in range(NCOMBO):
        b, f = combo // 2, combo % 2

        def _grp_cnt(g, _):
            bidx = idx_ref[combo * 2 + g // 8, pl.ds((g % 8) * 16, 16)]
            for j in range(16):
                m = bidx[j]
                plsc.addupdate(cnt.at[combo * M + m, pl.ds(0, 16)], ones16)
            return 0

        lax.fori_loop(0, P // 16, _grp_cnt, 0)

        pltpu.sync_copy(zeros_acc, acc)

        def _layer(l, _):
            pltpu.sync_copy(preds[f].at[b, l, pl.ds(p0, P), :], pbuf)

            def _grp(g, _):
                bidx = idx_ref[combo * 2 + g // 8, pl.ds((g % 8) * 16, 16)]
                for j in range(16):
                    m = bidx[j]
                    row = l * M + m
                    pr = g * 16 + j
                    for k in range(C // 16):
                        v = pbuf[pr, pl.ds(k * 16, 16)]
                        plsc.addupdate(acc.at[row, pl.ds(k * 16, 16)], v)
                    v = pbuf[pr, pl.ds(CTAIL, 16)]
                    v = jnp.where(tail_keep, v, 0.0)
                    plsc.addupdate(acc.at[row, pl.ds(CTAIL, 16)], v)
                return 0

            lax.fori_loop(0, P // 16, _grp, 0)
            return 0

        lax.fori_loop(0, L, _layer, 0)

        pltpu.sync_copy(acc, sums_out.at[wid, pl.ds(combo * L * M, L * M), :])

    pltpu.sync_copy(cnt, cnt_out.at[wid])


def _make_sc_kernel():
    mesh = plsc.VectorSubcoreMesh(core_axis_name="c", subcore_axis_name="s")
    return pl.kernel(
        _sc_kernel_body,
        out_type=[
            jax.ShapeDtypeStruct((NW, ROWS, C), jnp.float32),
            jax.ShapeDtypeStruct((NW, NCOMBO * M, 16), jnp.float32),
        ],
        mesh=mesh,
        scratch_types=[
            pltpu.VMEM((M, P), jnp.float32),
            pltpu.VMEM((NCOMBO * 2, 128), jnp.int32),
            pltpu.VMEM((P, C), jnp.float32),
            pltpu.VMEM((L * M, C), jnp.float32),
            pltpu.VMEM((NCOMBO * M, 16), jnp.float32),
        ],
    )


def _tc_body(s_ref, c_ref, o_ref):
    S = jnp.sum(s_ref[...], axis=0)
    K = jnp.sum(c_ref[...], axis=0)
    S4 = S.reshape(NCOMBO, L, M, C)
    cnt = K[:, 0:1].reshape(NCOMBO, 1, M, 1)
    denom = jnp.maximum(cnt, 1.0)
    fmap = jnp.where(cnt > 0, S4 / denom, 0.0)

    loss = jnp.zeros((L,), jnp.float32)
    nobj = jnp.zeros((), jnp.float32)
    for b in range(B):
        f1 = fmap[2 * b + 0]
        f2 = fmap[2 * b + 1]
        mask_obj = jnp.logical_and(jnp.sum(f1[0], axis=1) != 0,
                                   jnp.sum(f2[0], axis=1) != 0)
        maskf = mask_obj.astype(jnp.float32)
        t1 = f1 - jnp.max(f1, axis=2, keepdims=True)
        tgt = jnp.exp(t1)
        tgt = tgt / jnp.sum(tgt, axis=2, keepdims=True)
        t2 = f2 - jnp.max(f2, axis=2, keepdims=True)
        logp = t2 - jnp.log(jnp.sum(jnp.exp(t2), axis=2, keepdims=True))
        CE = -jnp.sum(tgt * logp, axis=2)
        loss = loss + jnp.sum(CE * maskf[None, :], axis=1) / jnp.maximum(
            jnp.sum(maskf), 1.0)
        nobj = nobj + jnp.sum(maskf)
    o_ref[...] = loss / jnp.maximum(nobj, 1.0)


def _tc_tail(sums, cnts):
    return pl.pallas_call(
        _tc_body,
        out_shape=jax.ShapeDtypeStruct((L,), jnp.float32),
    )(sums, cnts)


@jax.jit
def kernel(pred0, pred1, masks0, masks1):
    zeros_acc = jnp.zeros((L * M, C), jnp.float32)
    zeros_cnt = jnp.zeros((NCOMBO * M, 16), jnp.float32)
    sums, cnts = _make_sc_kernel()(pred0, pred1, masks0, masks1,
                                   zeros_acc, zeros_cnt)
    return _tc_tail(sums, cnts)
